# baseline (speedup 1.0000x reference)
import functools

import jax
import jax.numpy as jnp
from jax import lax
from jax.experimental import pallas as pl
from jax.experimental.pallas import tpu as pltpu
from jax.experimental.pallas import tpu_sc as plsc

L = 1024
VOCA_DIM = 64
ADD_DIM = 32
EMBED_DIM = VOCA_DIM + ADD_DIM
VOCAB = 100000
PACK_W = 128


_PT = 4096


def _pack_body(emb_ref, syn_ref, out_ref):
    z = jnp.zeros((_PT, PACK_W - EMBED_DIM), jnp.float32)
    out_ref[...] = jnp.concatenate([emb_ref[...], syn_ref[...], z], axis=1)


def _tc_pack(emb_weight, to_syn_weight):
    return pl.pallas_call(
        _pack_body,
        grid=(pl.cdiv(VOCAB, _PT),),
        in_specs=[
            pl.BlockSpec((_PT, VOCA_DIM), lambda i: (i, 0)),
            pl.BlockSpec((_PT, ADD_DIM), lambda i: (i, 0)),
        ],
        out_specs=pl.BlockSpec((_PT, PACK_W), lambda i: (i, 0)),
        out_shape=jax.ShapeDtypeStruct((VOCAB, PACK_W), jnp.float32),
    )(emb_weight, to_syn_weight)


_info = plsc.get_sparse_core_info()
_NC, _NS = _info.num_cores, _info.num_subcores
_NW = _NC * _NS
_B_PER_W = L // _NW


def _sc_gather(ids, packed):
    mesh = plsc.VectorSubcoreMesh(core_axis_name="c", subcore_axis_name="s")

    @functools.partial(
        pl.kernel,
        mesh=mesh,
        out_type=jax.ShapeDtypeStruct((L, PACK_W), jnp.float32),
        scratch_types=[
            pltpu.VMEM((_B_PER_W,), jnp.int32),
            pltpu.VMEM((_B_PER_W, PACK_W), jnp.float32),
            pltpu.SemaphoreType.DMA,
        ],
    )
    def gather_kernel(ids_hbm, tab_hbm, out_hbm, idx_v, rows_v, sem):
        wid = lax.axis_index("s") * _NC + lax.axis_index("c")
        base = wid * _B_PER_W
        pltpu.sync_copy(ids_hbm.at[pl.ds(base, _B_PER_W)], idx_v)
        pltpu.async_copy(tab_hbm.at[idx_v], rows_v, sem).wait()
        pltpu.sync_copy(rows_v, out_hbm.at[pl.ds(base, _B_PER_W)])

    return gather_kernel(ids, packed)


_VT = 2048
_NFULL = VOCAB // _VT
_VTAIL = VOCAB - _NFULL * _VT
_G = _NFULL + 1
_NBUF = 4


def _mm_body(rows_ref, synw_ref, pad_ref, rev_ref, out_hbm, bufs, tail_buf,
             sems, tail_sem):
    i = pl.program_id(0)
    slot = lax.rem(i, _NBUF)

    @pl.when(i >= _NBUF)
    def _wait_slot():
        pltpu.make_async_copy(
            bufs.at[slot],
            out_hbm.at[:, pl.ds((i - _NBUF) * _VT, _VT)],
            sems.at[slot],
        ).wait()

    emb = rows_ref[:, :VOCA_DIM]
    syn = rows_ref[:, VOCA_DIM:EMBED_DIM]
    proj = jnp.dot(syn, synw_ref[...], preferred_element_type=jnp.float32)
    x = jnp.concatenate([emb + proj, pad_ref[...]], axis=1)
    y = lax.dot_general(
        x, rev_ref[...],
        dimension_numbers=(((1,), (1,)), ((), ())),
        preferred_element_type=jnp.float32,
    )

    @pl.when(i < _G - 1)
    def _start_full():
        bufs[slot] = y
        pltpu.make_async_copy(
            bufs.at[slot],
            out_hbm.at[:, pl.ds(i * _VT, _VT)],
            sems.at[slot],
        ).start()

    @pl.when(i == _G - 1)
    def _tail_and_drain():
        tail_buf[...] = y[:, :_VTAIL]
        pltpu.make_async_copy(
            tail_buf,
            out_hbm.at[:, pl.ds(_NFULL * _VT, _VTAIL)],
            tail_sem,
        ).start()
        for k in range(1, _NBUF):
            s = (_G - 1 - k) % _NBUF
            pltpu.make_async_copy(
                bufs.at[s],
                out_hbm.at[:, pl.ds(0, _VT)],
                sems.at[s],
            ).wait()
        pltpu.make_async_copy(
            tail_buf,
            out_hbm.at[:, pl.ds(_NFULL * _VT, _VTAIL)],
            tail_sem,
        ).wait()


def _tc_matmul(rows, syn_weight, padding, rev_weight):
    return pl.pallas_call(
        _mm_body,
        grid=(_G,),
        in_specs=[
            pl.BlockSpec((L, PACK_W), lambda i: (0, 0)),
            pl.BlockSpec((ADD_DIM, VOCA_DIM), lambda i: (0, 0)),
            pl.BlockSpec((L, ADD_DIM), lambda i: (0, 0)),
            pl.BlockSpec((_VT, EMBED_DIM), lambda i: (i, 0)),
        ],
        out_specs=pl.BlockSpec(memory_space=pl.ANY),
        out_shape=jax.ShapeDtypeStruct((L, VOCAB), jnp.float32),
        scratch_shapes=[
            pltpu.VMEM((_NBUF, L, _VT), jnp.float32),
            pltpu.VMEM((L, _VTAIL), jnp.float32),
            pltpu.SemaphoreType.DMA((_NBUF,)),
            pltpu.SemaphoreType.DMA,
        ],
    )(rows, syn_weight, padding, rev_weight)


def kernel(ids, emb_weight, to_syn_weight, syn_weight, rev_weight, padding):
    packed = _tc_pack(emb_weight, to_syn_weight)
    rows = _sc_gather(ids, packed)
    return _tc_matmul(rows, syn_weight, padding[:L, :], rev_weight)


# speedup vs baseline: 3.2361x; 3.2361x over previous
import functools

import jax
import jax.numpy as jnp
from jax import lax
from jax.experimental import pallas as pl
from jax.experimental.pallas import tpu as pltpu
from jax.experimental.pallas import tpu_sc as plsc

L = 1024
VOCA_DIM = 64
ADD_DIM = 32
EMBED_DIM = VOCA_DIM + ADD_DIM
VOCAB = 100000
PACK_W = 128


_PT = 4096
_PG = pl.cdiv(VOCAB, _PT)
_PTAIL = VOCAB - (_PG - 1) * _PT
_PNBUF = 4


def _pack_body(embT_ref, synT_ref, out_hbm, bufs, sems):
    i = pl.program_id(0)
    slot = lax.rem(i, _PNBUF)

    @pl.when(i >= _PNBUF)
    def _wait_slot():
        pltpu.make_async_copy(
            bufs.at[slot], out_hbm.at[pl.ds((i - _PNBUF) * _PT, _PT)],
            sems.at[slot],
        ).wait()

    emb = embT_ref[...].T
    syn = synT_ref[...].T
    z = jnp.zeros((_PT, PACK_W - EMBED_DIM), jnp.float32)
    bufs[slot] = jnp.concatenate([emb, syn, z], axis=1)

    @pl.when(i < _PG - 1)
    def _start_full():
        pltpu.make_async_copy(
            bufs.at[slot], out_hbm.at[pl.ds(i * _PT, _PT)], sems.at[slot],
        ).start()

    @pl.when(i == _PG - 1)
    def _tail_and_drain():
        pltpu.make_async_copy(
            bufs.at[slot, pl.ds(0, _PTAIL)],
            out_hbm.at[pl.ds((_PG - 1) * _PT, _PTAIL)],
            sems.at[slot],
        ).start()
        for k in range(1, _PNBUF):
            s = (_PG - 1 - k) % _PNBUF
            pltpu.make_async_copy(
                bufs.at[s], out_hbm.at[pl.ds(0, _PT)], sems.at[s],
            ).wait()
        pltpu.make_async_copy(
            bufs.at[slot, pl.ds(0, _PTAIL)],
            out_hbm.at[pl.ds(0, _PTAIL)],
            sems.at[slot],
        ).wait()


def _tc_pack(embT, synT):
    return pl.pallas_call(
        _pack_body,
        grid=(_PG,),
        in_specs=[
            pl.BlockSpec((VOCA_DIM, _PT), lambda i: (0, i)),
            pl.BlockSpec((ADD_DIM, _PT), lambda i: (0, i)),
        ],
        out_specs=pl.BlockSpec(memory_space=pl.ANY),
        out_shape=jax.ShapeDtypeStruct((VOCAB, PACK_W), jnp.float32),
        scratch_shapes=[
            pltpu.VMEM((_PNBUF, _PT, PACK_W), jnp.float32),
            pltpu.SemaphoreType.DMA((_PNBUF,)),
        ],
    )(embT, synT)


_info = plsc.get_sparse_core_info()
_NC, _NS = _info.num_cores, _info.num_subcores
_NW = _NC * _NS
_B_PER_W = L // _NW


def _sc_gather(ids, packed):
    mesh = plsc.VectorSubcoreMesh(core_axis_name="c", subcore_axis_name="s")

    @functools.partial(
        pl.kernel,
        mesh=mesh,
        out_type=jax.ShapeDtypeStruct((L, PACK_W), jnp.float32),
        scratch_types=[
            pltpu.VMEM((_B_PER_W,), jnp.int32),
            pltpu.VMEM((_B_PER_W, PACK_W), jnp.float32),
            pltpu.SemaphoreType.DMA,
        ],
    )
    def gather_kernel(ids_hbm, tab_hbm, out_hbm, idx_v, rows_v, sem):
        wid = lax.axis_index("s") * _NC + lax.axis_index("c")
        base = wid * _B_PER_W
        pltpu.sync_copy(ids_hbm.at[pl.ds(base, _B_PER_W)], idx_v)
        pltpu.async_copy(tab_hbm.at[idx_v], rows_v, sem).wait()
        pltpu.sync_copy(rows_v, out_hbm.at[pl.ds(base, _B_PER_W)])

    return gather_kernel(ids, packed)


_VT = 2048
_G = pl.cdiv(VOCAB, _VT)
_VTAIL = VOCAB - (_G - 1) * _VT
_NBUF = 4


def _mm_body(rows_ref, synw_ref, padT_ref, revT_ref, out_hbm, bufs, sems):
    i = pl.program_id(0)
    slot = lax.rem(i, _NBUF)

    @pl.when(i >= _NBUF)
    def _wait_slot():
        pltpu.make_async_copy(
            bufs.at[slot], out_hbm.at[pl.ds((i - _NBUF) * _VT, _VT)],
            sems.at[slot],
        ).wait()

    rowsT = rows_ref[...].T
    embT = rowsT[:VOCA_DIM, :]
    synT = rowsT[VOCA_DIM:EMBED_DIM, :]
    projT = lax.dot_general(
        synw_ref[...], synT,
        dimension_numbers=(((0,), (0,)), ((), ())),
        preferred_element_type=jnp.float32,
    )
    xT = jnp.concatenate([embT + projT, padT_ref[...]], axis=0)
    bufs[slot] = lax.dot_general(
        revT_ref[...], xT,
        dimension_numbers=(((0,), (0,)), ((), ())),
        preferred_element_type=jnp.float32,
    )

    @pl.when(i < _G - 1)
    def _start_full():
        pltpu.make_async_copy(
            bufs.at[slot], out_hbm.at[pl.ds(i * _VT, _VT)], sems.at[slot],
        ).start()

    @pl.when(i == _G - 1)
    def _tail_and_drain():
        pltpu.make_async_copy(
            bufs.at[slot, pl.ds(0, _VTAIL)],
            out_hbm.at[pl.ds((_G - 1) * _VT, _VTAIL)],
            sems.at[slot],
        ).start()
        for k in range(1, _NBUF):
            s = (_G - 1 - k) % _NBUF
            pltpu.make_async_copy(
                bufs.at[s], out_hbm.at[pl.ds(0, _VT)], sems.at[s],
            ).wait()
        pltpu.make_async_copy(
            bufs.at[slot, pl.ds(0, _VTAIL)],
            out_hbm.at[pl.ds(0, _VTAIL)],
            sems.at[slot],
        ).wait()


def _tc_matmul(rows, syn_weight, padT, revT):
    return pl.pallas_call(
        _mm_body,
        grid=(_G,),
        in_specs=[
            pl.BlockSpec((L, PACK_W), lambda i: (0, 0)),
            pl.BlockSpec((ADD_DIM, VOCA_DIM), lambda i: (0, 0)),
            pl.BlockSpec((ADD_DIM, L), lambda i: (0, 0)),
            pl.BlockSpec((EMBED_DIM, _VT), lambda i: (0, i)),
        ],
        out_specs=pl.BlockSpec(memory_space=pl.ANY),
        out_shape=jax.ShapeDtypeStruct((VOCAB, L), jnp.float32),
        scratch_shapes=[
            pltpu.VMEM((_NBUF, _VT, L), jnp.float32),
            pltpu.SemaphoreType.DMA((_NBUF,)),
        ],
        compiler_params=pltpu.CompilerParams(
            fuse_transposed_lhs_in_matmul=True,
        ),
    )(rows, syn_weight, padT, revT)


def kernel(ids, emb_weight, to_syn_weight, syn_weight, rev_weight, padding):
    packed = _tc_pack(emb_weight.T, to_syn_weight.T)
    rows = _sc_gather(ids, packed)
    outT = _tc_matmul(rows, syn_weight, padding[:L, :].T, rev_weight.T)
    return outT.T
